# Initial kernel scaffold; baseline (speedup 1.0000x reference)
import jax
import jax.numpy as jnp
from jax.experimental import pallas as pl


def kernel(x, edge_index, W1, b1, W2, b2, W3, b3):
    raise NotImplementedError("write your pallas kernel here")


# speedup vs baseline: 2.4010x; 2.4010x over previous
import jax
import jax.numpy as jnp
from jax.experimental import pallas as pl


def _matmul_body(h_ref, w_ref, o_ref):
    o_ref[...] = jnp.dot(h_ref[...], w_ref[...],
                         preferred_element_type=jnp.float32)


def _mm(h, W):
    M, K = h.shape
    _, N = W.shape
    BM = 400
    return pl.pallas_call(
        _matmul_body,
        grid=(M // BM,),
        in_specs=[
            pl.BlockSpec((BM, K), lambda i: (i, 0)),
            pl.BlockSpec((K, N), lambda i: (0, 0)),
        ],
        out_specs=pl.BlockSpec((BM, N), lambda i: (i, 0)),
        out_shape=jax.ShapeDtypeStruct((M, N), jnp.float32),
    )(h, W)


def kernel(x, edge_index, W1, b1, W2, b2, W3, b3):
    N = x.shape[0]
    src = edge_index[0].astype(jnp.int32)
    dst = edge_index[1].astype(jnp.int32)

    deg = jax.ops.segment_sum(jnp.ones(src.shape, dtype=x.dtype), dst,
                              num_segments=N) + 1.0
    dis = jax.lax.rsqrt(deg)

    def conv(h, W, b):
        g = _mm(h, W) * dis[:, None]
        agg = jax.ops.segment_sum(g[src], dst, num_segments=N)
        return dis[:, None] * (agg + g) + b

    h = jax.nn.relu(conv(x, W1, b1))
    h = jax.nn.relu(conv(h, W2, b2))
    h = jax.nn.relu(conv(h, W3, b3))
    return h


# speedup vs baseline: 6.1413x; 2.5579x over previous
import functools

import jax
import jax.numpy as jnp
from jax import lax
from jax.experimental import pallas as pl
from jax.experimental.pallas import tpu as pltpu
from jax.experimental.pallas import tpu_sc as plsc

N_NODES = 10000
HALF = 5000
ACC_ROWS = 5120
CHUNK = 128
N_CHUNKS = 2512
CHUNKS_PER_SUBCORE = N_CHUNKS // 16
EDGES_PER_SUBCORE = CHUNKS_PER_SUBCORE * CHUNK
D = 128


def _sc_agg_body(deg_mode, g_hbm, src_hbm, dst_hbm, z_hbm, out_hbm,
                 srcv1d, dstv1d, srcv, ldstv, rows, acc, sem):
    c = lax.axis_index("c")
    s = lax.axis_index("s")
    base = pl.multiple_of(c * HALF, 8)
    trash = HALF + s

    acc_off = pl.multiple_of(s * (ACC_ROWS // 16), ACC_ROWS // 16)
    pltpu.sync_copy(z_hbm, acc.at[pl.ds(acc_off, ACC_ROWS // 16)])

    eoff = pl.multiple_of(s * EDGES_PER_SUBCORE, 128)
    if not deg_mode:
        pltpu.sync_copy(src_hbm.at[pl.ds(eoff, EDGES_PER_SUBCORE)], srcv1d)
    else:
        pltpu.sync_copy(g_hbm, rows)
    pltpu.sync_copy(dst_hbm.at[pl.ds(eoff, EDGES_PER_SUBCORE)], dstv1d)
    plsc.subcore_barrier()

    def chunk_body(i, carry):
        for k in range(CHUNK // 16):
            off = pl.multiple_of(i * CHUNK + k * 16, 16)
            if not deg_mode:
                srcv[pl.ds(k * 16, 16)] = srcv1d[pl.ds(off, 16)]
            ld = dstv1d[pl.ds(off, 16)] - base
            ok = (ld >= 0) & (ld < HALF)
            ldstv[pl.ds(k * 16, 16)] = jnp.where(ok, ld, trash)
        if not deg_mode:
            pltpu.async_copy(g_hbm.at[srcv], rows, sem).wait()
        pltpu.sync_copy(rows, acc.at[ldstv], add=True)
        return carry

    lax.fori_loop(0, CHUNKS_PER_SUBCORE, chunk_body, 0)
    plsc.subcore_barrier()

    @pl.when(s == 0)
    def _():
        pltpu.sync_copy(acc.at[pl.ds(0, HALF)], out_hbm.at[pl.ds(base, HALF)])


@functools.cache
def _sc_agg(deg_mode=False):
    mesh = plsc.VectorSubcoreMesh(core_axis_name="c", subcore_axis_name="s")
    return pl.kernel(
        functools.partial(_sc_agg_body, deg_mode),
        out_type=jax.ShapeDtypeStruct((N_NODES, D), jnp.float32),
        mesh=mesh,
        scratch_types=[
            pltpu.VMEM((EDGES_PER_SUBCORE,), jnp.int32),
            pltpu.VMEM((EDGES_PER_SUBCORE,), jnp.int32),
            pltpu.VMEM((CHUNK,), jnp.int32),
            pltpu.VMEM((CHUNK,), jnp.int32),
            pltpu.VMEM((CHUNK, D), jnp.float32),
            pltpu.VMEM_SHARED((ACC_ROWS, D), jnp.float32),
            pltpu.SemaphoreType.DMA,
        ],
    )


def _mm_scale_body(x_ref, w_ref, deg_ref, oa_ref, ob_ref):
    dis = lax.rsqrt(deg_ref[...] + 1.0)
    o = jnp.dot(x_ref[...], w_ref[...],
                preferred_element_type=jnp.float32) * dis
    oa_ref[...] = o[:, :128]
    ob_ref[...] = o[:, 128:]


def _mm_scale(x, W, deg_col):
    M, K = x.shape
    _, N = W.shape
    BM = 2000
    return pl.pallas_call(
        _mm_scale_body,
        grid=(M // BM,),
        in_specs=[
            pl.BlockSpec((BM, K), lambda i: (i, 0)),
            pl.BlockSpec((K, N), lambda i: (0, 0)),
            pl.BlockSpec((BM, 1), lambda i: (i, 0)),
        ],
        out_specs=[pl.BlockSpec((BM, 128), lambda i: (i, 0)),
                   pl.BlockSpec((BM, 128), lambda i: (i, 0))],
        out_shape=[jax.ShapeDtypeStruct((M, 128), jnp.float32),
                   jax.ShapeDtypeStruct((M, 128), jnp.float32)],
    )(x, W, deg_col)


def _fused_body(split_out, aa_ref, ab_ref, ga_ref, gb_ref, deg_ref, b_ref,
                w_ref, *o_refs):
    dis = lax.rsqrt(deg_ref[...] + 1.0)
    agg = jnp.concatenate([aa_ref[...], ab_ref[...]], axis=1)
    g = jnp.concatenate([ga_ref[...], gb_ref[...]], axis=1)
    h = jnp.maximum((agg + g) * dis + b_ref[...], 0.0)
    o = jnp.dot(h, w_ref[...], preferred_element_type=jnp.float32) * dis
    if split_out:
        o_refs[0][...] = o[:, :128]
        o_refs[1][...] = o[:, 128:]
    else:
        o_refs[0][...] = o


def _fused(agg_a, agg_b, g_a, g_b, deg_col, b, W):
    M = g_a.shape[0]
    K = 2 * g_a.shape[1]
    _, N = W.shape
    BM = 2000
    split_out = N == 256
    n_out = 2 if split_out else 1
    return pl.pallas_call(
        functools.partial(_fused_body, split_out),
        grid=(M // BM,),
        in_specs=[
            pl.BlockSpec((BM, 128), lambda i: (i, 0)),
            pl.BlockSpec((BM, 128), lambda i: (i, 0)),
            pl.BlockSpec((BM, 128), lambda i: (i, 0)),
            pl.BlockSpec((BM, 128), lambda i: (i, 0)),
            pl.BlockSpec((BM, 1), lambda i: (i, 0)),
            pl.BlockSpec((1, K), lambda i: (0, 0)),
            pl.BlockSpec((K, N), lambda i: (0, 0)),
        ],
        out_specs=[pl.BlockSpec((BM, 128), lambda i: (i, 0))] * n_out,
        out_shape=[jax.ShapeDtypeStruct((M, 128), jnp.float32)] * n_out,
    )(agg_a, agg_b, g_a, g_b, deg_col, b.reshape(1, K), W)


def _final_body(agg_ref, g_ref, deg_ref, b_ref, o_ref):
    dis = lax.rsqrt(deg_ref[...] + 1.0)
    o_ref[...] = jnp.maximum(
        (agg_ref[...] + g_ref[...]) * dis + b_ref[...], 0.0)


def _final(agg, g, deg_col, b):
    M, K = g.shape
    BM = 2000
    return pl.pallas_call(
        _final_body,
        grid=(M // BM,),
        in_specs=[
            pl.BlockSpec((BM, K), lambda i: (i, 0)),
            pl.BlockSpec((BM, K), lambda i: (i, 0)),
            pl.BlockSpec((BM, 1), lambda i: (i, 0)),
            pl.BlockSpec((1, K), lambda i: (0, 0)),
        ],
        out_specs=pl.BlockSpec((BM, K), lambda i: (i, 0)),
        out_shape=jax.ShapeDtypeStruct((M, K), jnp.float32),
    )(agg, g, deg_col, b.reshape(1, K))


def kernel(x, edge_index, W1, b1, W2, b2, W3, b3):
    src = edge_index[0].astype(jnp.int32)
    dst = edge_index[1].astype(jnp.int32)
    pad = N_CHUNKS * CHUNK - src.shape[0]
    src1d = jnp.concatenate([src, jnp.zeros((pad,), jnp.int32)])
    dst1d = jnp.concatenate([dst, jnp.full((pad,), N_NODES, jnp.int32)])

    z = jnp.zeros((ACC_ROWS // 16, D), jnp.float32)
    ones = jnp.ones((CHUNK, D), jnp.float32)

    deg128 = _sc_agg(True)(ones, src1d, dst1d, z)
    deg_col = deg128[:, :1]

    agg = _sc_agg(False)
    g1a, g1b = _mm_scale(x, W1, deg_col)
    agg1a = agg(g1a, src1d, dst1d, z)
    agg1b = agg(g1b, src1d, dst1d, z)
    g2a, g2b = _fused(agg1a, agg1b, g1a, g1b, deg_col, b1, W2)
    agg2a = agg(g2a, src1d, dst1d, z)
    agg2b = agg(g2b, src1d, dst1d, z)
    g3 = _fused(agg2a, agg2b, g2a, g2b, deg_col, b2, W3)[0]
    agg3 = agg(g3, src1d, dst1d, z)
    return _final(agg3, g3, deg_col, b3)
